# baseline (speedup 1.0000x reference)
import functools

import jax
import jax.numpy as jnp
from jax.experimental import pallas as pl
from jax.experimental.pallas import tpu as pltpu
from jax.experimental.pallas import tpu_sc as plsc

_HID = 256
_K = 6
_NC = 512
_NF = 4096
_FBLK = 256
_SBLK = 512

_HI = jax.lax.Precision.HIGHEST


def _gelu(x):
    return x * 0.5 * (1.0 + jax.lax.erf(x * 0.7071067811865476))


def _mm(a, b, precision=None):
    return jax.lax.dot_general(
        a, b, (((1,), (0,)), ((), ())),
        precision=precision, preferred_element_type=jnp.float32)


def _b16(x):
    return x.astype(jnp.bfloat16).astype(jnp.float32)


def _d2_block(ci_ref, ct_ref, row0):
    xi = ci_ref[:, 0:1]
    yi = ci_ref[:, 1:2]
    xj = ct_ref[0:1, :]
    yj = ct_ref[1:2, :]
    si = xi * xi + yi * yi
    sj = xj * xj + yj * yj
    d2 = si + sj - 2.0 * (_b16(xi) * _b16(xj) + _b16(yi) * _b16(yj))
    d2 = jnp.maximum(d2, 0.0)
    col = jax.lax.broadcasted_iota(jnp.int32, d2.shape, 1)
    row = jax.lax.broadcasted_iota(jnp.int32, d2.shape, 0) + row0
    return jnp.where(col == row, 1e9, d2)


def _topk_rows(d2, n_cols, k):
    col = jax.lax.broadcasted_iota(jnp.int32, d2.shape, 1)
    idxs, vals = [], []
    for _ in range(k):
        m = jnp.min(d2, axis=1, keepdims=True)
        cand = jnp.where(d2 == m, col, 2 * n_cols)
        a = jnp.min(cand, axis=1, keepdims=True)
        idxs.append(a)
        vals.append(m)
        d2 = jnp.where(col == a, 1e9, d2)
    return idxs, vals


def _topk6_fold(d2, n_cols):
    grp = n_cols // 8
    big = jnp.float32(3e9)
    sub = d2.shape[0]
    col = jax.lax.broadcasted_iota(jnp.int32, (sub, grp), 1)
    v1 = v2 = v3 = jnp.full((sub, grp), big, jnp.float32)
    c1 = c2 = c3 = jnp.zeros((sub, grp), jnp.int32)
    for g in range(8):
        v = d2[:, g * grp:(g + 1) * grp]
        c = col + g * grp
        lt1 = v < v1
        lt2 = v < v2
        lt3 = v < v3
        v3 = jnp.where(lt2, v2, jnp.where(lt3, v, v3))
        c3 = jnp.where(lt2, c2, jnp.where(lt3, c, c3))
        v2 = jnp.where(lt1, v1, jnp.where(lt2, v, v2))
        c2 = jnp.where(lt1, c1, jnp.where(lt2, c, c2))
        v1 = jnp.where(lt1, v, v1)
        c1 = jnp.where(lt1, c, c1)
    vcat = jnp.concatenate([v1, v2, v3], axis=1)
    ccat = jnp.concatenate([c1, c2, c3], axis=1)
    idxs, vals = [], []
    for _ in range(_K):
        m = jnp.min(vcat, axis=1, keepdims=True)
        cand = jnp.where(vcat == m, ccat, 2 * n_cols)
        a = jnp.min(cand, axis=1, keepdims=True)
        idxs.append(a)
        vals.append(m)
        vcat = jnp.where(ccat == a, big, vcat)
    return idxs, vals


def _knn_weights(vals):
    inv = [1.0 / jnp.maximum(jnp.sqrt(v), 1e-4) for v in vals]
    tot = functools.reduce(jnp.add, inv)
    tot = jnp.maximum(tot, 1e-8)
    return [x / tot for x in inv]


def _enc_knn_f_body(x_ref, w_ref, b_ref, ci_ref, ct_ref,
                    h_ref, h16_ref, idx_ref, w_out_ref):
    h = _gelu(_mm(x_ref[...], w_ref[...]) + b_ref[...])
    h_ref[...] = h
    h16_ref[...] = h.astype(jnp.bfloat16)
    d2 = _d2_block(ci_ref, ct_ref, pl.program_id(0) * _FBLK)
    idxs, vals = _topk6_fold(d2, _NF)
    ws = _knn_weights(vals)
    for k in range(_K):
        idx_ref[:, k:k + 1] = idxs[k]
        w_out_ref[:, k:k + 1] = ws[k]
    idx_ref[:, _K:_K + 2] = jnp.zeros((_FBLK, 2), jnp.int32)
    w_out_ref[:, _K:_K + 2] = jnp.zeros((_FBLK, 2), jnp.float32)


def _enc_knn_fine(f_feats, W, b, f_cents, f_cents_t):
    return pl.pallas_call(
        _enc_knn_f_body,
        grid=(_NF // _FBLK,),
        in_specs=[
            pl.BlockSpec((_FBLK, _HID), lambda i: (i, 0)),
            pl.BlockSpec((_HID, _HID), lambda i: (0, 0)),
            pl.BlockSpec((1, _HID), lambda i: (0, 0)),
            pl.BlockSpec((_FBLK, 2), lambda i: (i, 0)),
            pl.BlockSpec((2, _NF), lambda i: (0, 0)),
        ],
        out_specs=[
            pl.BlockSpec((_FBLK, _HID), lambda i: (i, 0)),
            pl.BlockSpec((_FBLK, _HID), lambda i: (i, 0)),
            pl.BlockSpec((_FBLK, 8), lambda i: (i, 0)),
            pl.BlockSpec((_FBLK, 8), lambda i: (i, 0)),
        ],
        out_shape=[
            jax.ShapeDtypeStruct((_NF, _HID), jnp.float32),
            jax.ShapeDtypeStruct((_NF, _HID), jnp.bfloat16),
            jax.ShapeDtypeStruct((_NF, 8), jnp.int32),
            jax.ShapeDtypeStruct((_NF, 8), jnp.float32),
        ],
    )(f_feats, W, b, f_cents, f_cents_t)


def _ln(x, g, b):
    m = x.mean(-1, keepdims=True)
    v = ((x - m) ** 2).mean(-1, keepdims=True)
    return (x - m) / jnp.sqrt(v + 1e-5) * g + b


def _coarse_a_body(cf_ref, cc_ref, cct_ref, encW_ref, encb_ref,
                   ws_ref, wn_ref, g_ref, b_ref, hc_ref):
    hc0 = _gelu(_mm(cf_ref[...], encW_ref[...]) + encb_ref[...])
    d2 = _d2_block(cc_ref, cct_ref, 0)
    idxs, vals = _topk_rows(d2, _NC, _K)
    ws = _knn_weights(vals)
    col = jax.lax.broadcasted_iota(jnp.int32, (_NC, _NC), 1)
    A = jnp.zeros((_NC, _NC), jnp.float32)
    for k in range(_K):
        A = A + jnp.where(col == idxs[k], ws[k], 0.0)
    agg = _mm(A, hc0, precision=_HI)
    out = _gelu(_mm(hc0, ws_ref[...]) + _mm(agg, wn_ref[...]))
    hc_ref[...] = hc0 + _ln(out, g_ref[...], b_ref[...])


def _coarse_a(c_feats, c_cents, c_cents_t, p):
    full = lambda shape: pl.BlockSpec(shape, lambda: (0,) * len(shape))
    return pl.pallas_call(
        _coarse_a_body,
        in_specs=[
            full((_NC, _HID)), full((_NC, 2)), full((2, _NC)),
            full((_HID, _HID)), full((1, _HID)),
            full((_HID, _HID)), full((_HID, _HID)),
            full((1, _HID)), full((1, _HID)),
        ],
        out_specs=full((_NC, _HID)),
        out_shape=jax.ShapeDtypeStruct((_NC, _HID), jnp.float32),
    )(c_feats, c_cents, c_cents_t,
      p['enc_c_W'], p['enc_c_b'].reshape(1, -1),
      p['gc_ws'], p['gc_wn'], p['gc_g'].reshape(1, -1), p['gc_b'].reshape(1, -1))


def _sc_gather(values, indices):
    M = indices.shape[1]
    D = values.shape[1]
    window = 128
    mesh = plsc.VectorSubcoreMesh(core_axis_name="core",
                                  subcore_axis_name="subcore")

    @functools.partial(
        pl.kernel,
        out_type=jax.ShapeDtypeStruct((M, D), values.dtype),
        mesh=mesh)
    def gather_kernel(x_hbm, i_hbm, o_hbm):
        def body(i_vmem, o_vmem):
            pltpu.sync_copy(x_hbm.at[i_vmem.at[0]], o_vmem)

        pltpu.emit_pipeline(
            body,
            grid=(M // window,),
            in_specs=[pl.BlockSpec((1, window), lambda i: (0, i))],
            out_specs=[pl.BlockSpec((window, D), lambda i: (i, 0))],
            core_axis_name=("core", "subcore"),
            dimension_semantics=(pltpu.PARALLEL,),
        )(i_hbm, o_hbm)

    return gather_kernel(values, indices)


def _sage_f_body(h_ref, nei_ref, w_ref, par_ref, ones_ref, ws_ref, wn_ref,
                 g_ref, b_ref, f2cW_ref, f2cb_ref, hf_ref, agg_ref, cnt_ref):
    @pl.when(pl.program_id(0) == 0)
    def _():
        agg_ref[...] = jnp.zeros_like(agg_ref)
        cnt_ref[...] = jnp.zeros_like(cnt_ref)

    agg = w_ref[:, 0:1] * nei_ref[:, 0, :].astype(jnp.float32)
    for k in range(1, _K):
        agg = agg + w_ref[:, k:k + 1] * nei_ref[:, k, :].astype(jnp.float32)
    h = h_ref[...]
    out = _gelu(_mm(h, ws_ref[...]) + _mm(agg, wn_ref[...]))
    hf1 = h + _ln(out, g_ref[...], b_ref[...])
    hf_ref[...] = hf1
    msg = _mm(hf1, f2cW_ref[...]) + f2cb_ref[...]
    p_col = par_ref[0]
    cols = jax.lax.broadcasted_iota(jnp.int32, (1, _NC), 1)
    P = jnp.where(p_col == cols, 1.0, 0.0).astype(jnp.float32)
    agg_ref[...] += jax.lax.dot_general(
        P, msg, (((0,), (0,)), ((), ())),
        preferred_element_type=jnp.float32)
    cnt_ref[...] += jax.lax.dot_general(
        P, ones_ref[...], (((0,), (0,)), ((), ())),
        preferred_element_type=jnp.float32)


def _sage_fine(hf0, nei, wf, par3, p):
    full0 = lambda shape: pl.BlockSpec(shape, lambda i: (0,) * len(shape))
    ones = jnp.ones((_FBLK, 128), jnp.float32)
    return pl.pallas_call(
        _sage_f_body,
        grid=(_NF // _FBLK,),
        in_specs=[
            pl.BlockSpec((_FBLK, _HID), lambda i: (i, 0)),
            pl.BlockSpec((_FBLK, _K, _HID), lambda i: (i, 0, 0)),
            pl.BlockSpec((_FBLK, 8), lambda i: (i, 0)),
            pl.BlockSpec((1, _FBLK, 1), lambda i: (i, 0, 0)),
            full0((_FBLK, 128)),
            full0((_HID, _HID)), full0((_HID, _HID)),
            full0((1, _HID)), full0((1, _HID)),
            full0((_HID, _HID)), full0((1, _HID)),
        ],
        out_specs=[
            pl.BlockSpec((_FBLK, _HID), lambda i: (i, 0)),
            pl.BlockSpec((_NC, _HID), lambda i: (0, 0)),
            pl.BlockSpec((_NC, 128), lambda i: (0, 0)),
        ],
        out_shape=[
            jax.ShapeDtypeStruct((_NF, _HID), jnp.float32),
            jax.ShapeDtypeStruct((_NC, _HID), jnp.float32),
            jax.ShapeDtypeStruct((_NC, 128), jnp.float32),
        ],
    )(hf0, nei, wf, par3, ones,
      p['gf_ws'], p['gf_wn'], p['gf_g'].reshape(1, -1), p['gf_b'].reshape(1, -1),
      p['f2c_W'], p['f2c_b'].reshape(1, -1))


def _parent_body(ff_ref, cfm_ref, fc_ref, cct_ref, par_ref):
    ff = ff_ref[...]
    inter = jax.lax.dot_general(
        ff, cfm_ref[...], (((1,), (1,)), ((), ())),
        preferred_element_type=jnp.float32)
    area = jnp.maximum(
        jnp.sum(ff.astype(jnp.float32), axis=1, keepdims=True), 1.0)
    ratio = inter / area
    mx = jnp.max(ratio, axis=1, keepdims=True)
    col = jax.lax.broadcasted_iota(jnp.int32, ratio.shape, 1)
    best = jnp.min(jnp.where(ratio == mx, col, 2 * _NC), axis=1, keepdims=True)
    xi = fc_ref[:, 0:1]
    yi = fc_ref[:, 1:2]
    xj = cct_ref[0:1, :]
    yj = cct_ref[1:2, :]
    d2 = (xi * xi + yi * yi) + (xj * xj + yj * yj) \
        - 2.0 * (_b16(xi) * _b16(xj) + _b16(yi) * _b16(yj))
    d2 = jnp.maximum(d2, 0.0)
    mn = jnp.min(d2, axis=1, keepdims=True)
    near = jnp.min(jnp.where(d2 == mn, col, 2 * _NC), axis=1, keepdims=True)
    par_ref[0] = jnp.where(mx > 0.5, best, near)


def _parent(ff, cfm, f_cents, c_cents_t):
    return pl.pallas_call(
        _parent_body,
        grid=(_NF // _FBLK,),
        in_specs=[
            pl.BlockSpec((_FBLK, _NF), lambda i: (i, 0)),
            pl.BlockSpec((_NC, _NF), lambda i: (0, 0)),
            pl.BlockSpec((_FBLK, 2), lambda i: (i, 0)),
            pl.BlockSpec((2, _NC), lambda i: (0, 0)),
        ],
        out_specs=pl.BlockSpec((1, _FBLK, 1), lambda i: (i, 0, 0)),
        out_shape=jax.ShapeDtypeStruct((_NF // _FBLK, _FBLK, 1), jnp.int32),
    )(ff, cfm, f_cents, c_cents_t)


def _heads_body(hf_ref, par_ref, hc1_ref, agg_ref, cnt_ref,
                ccW1_ref, ccb1_ref, ccW2_ref, ccb2_ref,
                c2fW_ref, c2fb_ref, cfW1_ref, cfb1_ref, cfW2_ref, cfb2_ref,
                pc_ref, pf_ref, hc2_ref):
    i = pl.program_id(0)

    @pl.when(i == 0)
    def _():
        hc2 = hc1_ref[...] + agg_ref[...] / jnp.maximum(cnt_ref[:, 0:1], 1.0)
        hc2_ref[...] = hc2
        t = _gelu(_mm(hc2, ccW1_ref[...]) + ccb1_ref[...])
        pc_ref[...] = jax.nn.sigmoid(_mm(t, ccW2_ref[...]) + ccb2_ref[...])

    @pl.when(i > 0)
    def _():
        p_col = par_ref[0]
        cols = jax.lax.broadcasted_iota(jnp.int32, (1, _NC), 1)
        P = jnp.where(p_col == cols, 1.0, 0.0).astype(jnp.float32)
        g = _mm(P, hc2_ref[...])
        hf2 = hf_ref[...] + _mm(g, c2fW_ref[...]) + c2fb_ref[...]
        t = _gelu(_mm(hf2, cfW1_ref[...]) + cfb1_ref[...])
        pf_ref[...] = jax.nn.sigmoid(_mm(t, cfW2_ref[...]) + cfb2_ref[...])


def _heads(hf1, par3, hc1, agg, cnt, p):
    full0 = lambda shape: pl.BlockSpec(shape, lambda i: (0,) * len(shape))
    prev = lambda i: jnp.maximum(i - 1, 0)
    pc, pf = pl.pallas_call(
        _heads_body,
        grid=(_NF // _FBLK + 1,),
        in_specs=[
            pl.BlockSpec((_FBLK, _HID), lambda i: (prev(i), 0)),
            pl.BlockSpec((1, _FBLK, 1), lambda i: (prev(i), 0, 0)),
            full0((_NC, _HID)), full0((_NC, _HID)), full0((_NC, 128)),
            full0((_HID, _HID // 2)), full0((1, _HID // 2)),
            full0((_HID // 2, 1)), full0((1, 1)),
            full0((_HID, _HID)), full0((1, _HID)),
            full0((_HID, _HID // 2)), full0((1, _HID // 2)),
            full0((_HID // 2, 1)), full0((1, 1)),
        ],
        out_specs=[
            full0((_NC, 1)),
            pl.BlockSpec((_FBLK, 1), lambda i: (prev(i), 0)),
        ],
        out_shape=[
            jax.ShapeDtypeStruct((_NC, 1), jnp.float32),
            jax.ShapeDtypeStruct((_NF, 1), jnp.float32),
        ],
        scratch_shapes=[pltpu.VMEM((_NC, _HID), jnp.float32)],
    )(hf1, par3, hc1, agg, cnt,
      p['cc_W1'], p['cc_b1'].reshape(1, -1), p['cc_W2'], p['cc_b2'].reshape(1, -1),
      p['c2f_W'], p['c2f_b'].reshape(1, -1),
      p['cf_W1'], p['cf_b1'].reshape(1, -1), p['cf_W2'], p['cf_b2'].reshape(1, -1))
    return pc, pf


def kernel(c_feats, c_cents, c_masks, f_feats, f_cents, f_masks, params):
    p = params
    f_cents_t = f_cents.T
    c_cents_t = c_cents.T
    ff = f_masks.reshape(_NF, -1)
    cfm = c_masks.reshape(_NC, -1)

    hf0, hf16, idxf, wf = _enc_knn_fine(
        f_feats, p['enc_f_W'], p['enc_f_b'].reshape(1, -1), f_cents, f_cents_t)
    hf16_i32 = jax.lax.bitcast_convert_type(
        hf16.reshape(_NF, _HID // 2, 2), jnp.int32)
    nei_i32 = _sc_gather(hf16_i32, idxf[:, :_K].reshape(1, _NF * _K))
    nei_flat = jax.lax.bitcast_convert_type(
        nei_i32, jnp.bfloat16).reshape(_NF * _K, _HID)
    par3 = _parent(ff, cfm, f_cents, c_cents_t)
    hc1 = _coarse_a(c_feats, c_cents, c_cents_t, p)
    nei = nei_flat.reshape(_NF, _K, _HID)
    hf1, agg, cnt = _sage_fine(hf0, nei, wf, par3, p)
    pc, pf = _heads(hf1, par3, hc1, agg, cnt, p)
    return pc.reshape(_NC), pf.reshape(_NF)


# speedup vs baseline: 3.2586x; 3.2586x over previous
import functools

import jax
import jax.numpy as jnp
from jax.experimental import pallas as pl
from jax.experimental.pallas import tpu as pltpu
from jax.experimental.pallas import tpu_sc as plsc

_HID = 256
_K = 6
_NC = 512
_NF = 4096
_FBLK = 256
_SBLK = 512

_HI = jax.lax.Precision.HIGHEST


def _gelu(x):
    return x * 0.5 * (1.0 + jax.lax.erf(x * 0.7071067811865476))


def _mm(a, b, precision=None):
    return jax.lax.dot_general(
        a, b, (((1,), (0,)), ((), ())),
        precision=precision, preferred_element_type=jnp.float32)


def _b16(x):
    return x.astype(jnp.bfloat16).astype(jnp.float32)


def _d2_block(ci_ref, ct_ref, row0):
    xi = ci_ref[:, 0:1]
    yi = ci_ref[:, 1:2]
    xj = ct_ref[0:1, :]
    yj = ct_ref[1:2, :]
    si = xi * xi + yi * yi
    sj = xj * xj + yj * yj
    d2 = si + sj - 2.0 * (_b16(xi) * _b16(xj) + _b16(yi) * _b16(yj))
    d2 = jnp.maximum(d2, 0.0)
    col = jax.lax.broadcasted_iota(jnp.int32, d2.shape, 1)
    row = jax.lax.broadcasted_iota(jnp.int32, d2.shape, 0) + row0
    return jnp.where(col == row, 1e9, d2)


def _topk_rows(d2, n_cols, k):
    col = jax.lax.broadcasted_iota(jnp.int32, d2.shape, 1)
    idxs, vals = [], []
    for _ in range(k):
        m = jnp.min(d2, axis=1, keepdims=True)
        cand = jnp.where(d2 == m, col, 2 * n_cols)
        a = jnp.min(cand, axis=1, keepdims=True)
        idxs.append(a)
        vals.append(m)
        d2 = jnp.where(col == a, 1e9, d2)
    return idxs, vals


def _topk6_fold(d2, n_cols):
    grp = n_cols // 8
    big = jnp.float32(3e9)
    sub = d2.shape[0]
    col = jax.lax.broadcasted_iota(jnp.int32, (sub, grp), 1)
    v1 = v2 = v3 = jnp.full((sub, grp), big, jnp.float32)
    c1 = c2 = c3 = jnp.zeros((sub, grp), jnp.int32)
    for g in range(8):
        v = d2[:, g * grp:(g + 1) * grp]
        c = col + g * grp
        lt1 = v < v1
        lt2 = v < v2
        lt3 = v < v3
        v3 = jnp.where(lt2, v2, jnp.where(lt3, v, v3))
        c3 = jnp.where(lt2, c2, jnp.where(lt3, c, c3))
        v2 = jnp.where(lt1, v1, jnp.where(lt2, v, v2))
        c2 = jnp.where(lt1, c1, jnp.where(lt2, c, c2))
        v1 = jnp.where(lt1, v, v1)
        c1 = jnp.where(lt1, c, c1)
    vcat = jnp.concatenate([v1, v2, v3], axis=1)
    ccat = jnp.concatenate([c1, c2, c3], axis=1)
    idxs, vals = [], []
    for _ in range(_K):
        m = jnp.min(vcat, axis=1, keepdims=True)
        cand = jnp.where(vcat == m, ccat, 2 * n_cols)
        a = jnp.min(cand, axis=1, keepdims=True)
        idxs.append(a)
        vals.append(m)
        vcat = jnp.where(ccat == a, big, vcat)
    return idxs, vals


def _knn_weights(vals):
    inv = [1.0 / jnp.maximum(jnp.sqrt(v), 1e-4) for v in vals]
    tot = functools.reduce(jnp.add, inv)
    tot = jnp.maximum(tot, 1e-8)
    return [x / tot for x in inv]


def _enc_knn_f_body(x_ref, w_ref, b_ref, ci_ref, ct_ref,
                    h_ref, idx_ref, w_out_ref):
    h_ref[...] = _gelu(_mm(x_ref[...], w_ref[...]) + b_ref[...])
    d2 = _d2_block(ci_ref, ct_ref, pl.program_id(0) * _FBLK)
    idxs, vals = _topk6_fold(d2, _NF)
    ws = _knn_weights(vals)
    for k in range(_K):
        idx_ref[:, k:k + 1] = idxs[k]
        w_out_ref[:, k:k + 1] = ws[k]
    idx_ref[:, _K:_K + 2] = jnp.zeros((_FBLK, 2), jnp.int32)
    w_out_ref[:, _K:_K + 2] = jnp.zeros((_FBLK, 2), jnp.float32)


def _enc_knn_fine(f_feats, W, b, f_cents, f_cents_t):
    return pl.pallas_call(
        _enc_knn_f_body,
        grid=(_NF // _FBLK,),
        in_specs=[
            pl.BlockSpec((_FBLK, _HID), lambda i: (i, 0)),
            pl.BlockSpec((_HID, _HID), lambda i: (0, 0)),
            pl.BlockSpec((1, _HID), lambda i: (0, 0)),
            pl.BlockSpec((_FBLK, 2), lambda i: (i, 0)),
            pl.BlockSpec((2, _NF), lambda i: (0, 0)),
        ],
        out_specs=[
            pl.BlockSpec((_FBLK, _HID), lambda i: (i, 0)),
            pl.BlockSpec((_FBLK, 8), lambda i: (i, 0)),
            pl.BlockSpec((_FBLK, 8), lambda i: (i, 0)),
        ],
        out_shape=[
            jax.ShapeDtypeStruct((_NF, _HID), jnp.float32),
            jax.ShapeDtypeStruct((_NF, 8), jnp.int32),
            jax.ShapeDtypeStruct((_NF, 8), jnp.float32),
        ],
    )(f_feats, W, b, f_cents, f_cents_t)


def _ln(x, g, b):
    m = x.mean(-1, keepdims=True)
    v = ((x - m) ** 2).mean(-1, keepdims=True)
    return (x - m) / jnp.sqrt(v + 1e-5) * g + b


def _coarse_a_body(cf_ref, cc_ref, cct_ref, encW_ref, encb_ref,
                   ws_ref, wn_ref, g_ref, b_ref, hc_ref):
    hc0 = _gelu(_mm(cf_ref[...], encW_ref[...]) + encb_ref[...])
    d2 = _d2_block(cc_ref, cct_ref, 0)
    idxs, vals = _topk_rows(d2, _NC, _K)
    ws = _knn_weights(vals)
    col = jax.lax.broadcasted_iota(jnp.int32, (_NC, _NC), 1)
    A = jnp.zeros((_NC, _NC), jnp.float32)
    for k in range(_K):
        A = A + jnp.where(col == idxs[k], ws[k], 0.0)
    agg = _mm(A, hc0, precision=_HI)
    out = _gelu(_mm(hc0, ws_ref[...]) + _mm(agg, wn_ref[...]))
    hc_ref[...] = hc0 + _ln(out, g_ref[...], b_ref[...])


def _coarse_a(c_feats, c_cents, c_cents_t, p):
    full = lambda shape: pl.BlockSpec(shape, lambda: (0,) * len(shape))
    return pl.pallas_call(
        _coarse_a_body,
        in_specs=[
            full((_NC, _HID)), full((_NC, 2)), full((2, _NC)),
            full((_HID, _HID)), full((1, _HID)),
            full((_HID, _HID)), full((_HID, _HID)),
            full((1, _HID)), full((1, _HID)),
        ],
        out_specs=full((_NC, _HID)),
        out_shape=jax.ShapeDtypeStruct((_NC, _HID), jnp.float32),
    )(c_feats, c_cents, c_cents_t,
      p['enc_c_W'], p['enc_c_b'].reshape(1, -1),
      p['gc_ws'], p['gc_wn'], p['gc_g'].reshape(1, -1), p['gc_b'].reshape(1, -1))


def _sc_gather(values, indices):
    M = indices.shape[1]
    D = values.shape[1]
    window = 128
    mesh = plsc.VectorSubcoreMesh(core_axis_name="core",
                                  subcore_axis_name="subcore")

    @functools.partial(
        pl.kernel,
        out_type=jax.ShapeDtypeStruct((M, D), values.dtype),
        mesh=mesh)
    def gather_kernel(x_hbm, i_hbm, o_hbm):
        def body(i_vmem, o_vmem):
            pltpu.sync_copy(x_hbm.at[i_vmem.at[0]], o_vmem)

        pltpu.emit_pipeline(
            body,
            grid=(M // window,),
            in_specs=[pl.BlockSpec((1, window), lambda i: (0, i))],
            out_specs=[pl.BlockSpec((window, D), lambda i: (i, 0))],
            core_axis_name=("core", "subcore"),
            dimension_semantics=(pltpu.PARALLEL,),
        )(i_hbm, o_hbm)

    return gather_kernel(values, indices)


def _sage_f_body(h_ref, nei_ref, w_ref, par_ref, ones_ref, ws_ref, wn_ref,
                 g_ref, b_ref, f2cW_ref, f2cb_ref, hf_ref, agg_ref, cnt_ref):
    @pl.when(pl.program_id(0) == 0)
    def _():
        agg_ref[...] = jnp.zeros_like(agg_ref)
        cnt_ref[...] = jnp.zeros_like(cnt_ref)

    agg = w_ref[:, 0:1] * nei_ref[:, 0, :]
    for k in range(1, _K):
        agg = agg + w_ref[:, k:k + 1] * nei_ref[:, k, :]
    h = h_ref[...]
    out = _gelu(_mm(h, ws_ref[...]) + _mm(agg, wn_ref[...]))
    hf1 = h + _ln(out, g_ref[...], b_ref[...])
    hf_ref[...] = hf1
    msg = _mm(hf1, f2cW_ref[...]) + f2cb_ref[...]
    p_col = par_ref[0]
    cols = jax.lax.broadcasted_iota(jnp.int32, (1, _NC), 1)
    P = jnp.where(p_col == cols, 1.0, 0.0).astype(jnp.float32)
    agg_ref[...] += jax.lax.dot_general(
        P, msg, (((0,), (0,)), ((), ())),
        preferred_element_type=jnp.float32)
    cnt_ref[...] += jax.lax.dot_general(
        P, ones_ref[...], (((0,), (0,)), ((), ())),
        preferred_element_type=jnp.float32)


def _sage_fine(hf0, nei, wf, par3, p):
    full0 = lambda shape: pl.BlockSpec(shape, lambda i: (0,) * len(shape))
    ones = jnp.ones((_FBLK, 128), jnp.float32)
    return pl.pallas_call(
        _sage_f_body,
        grid=(_NF // _FBLK,),
        in_specs=[
            pl.BlockSpec((_FBLK, _HID), lambda i: (i, 0)),
            pl.BlockSpec((_FBLK, _K, _HID), lambda i: (i, 0, 0)),
            pl.BlockSpec((_FBLK, 8), lambda i: (i, 0)),
            pl.BlockSpec((1, _FBLK, 1), lambda i: (i, 0, 0)),
            full0((_FBLK, 128)),
            full0((_HID, _HID)), full0((_HID, _HID)),
            full0((1, _HID)), full0((1, _HID)),
            full0((_HID, _HID)), full0((1, _HID)),
        ],
        out_specs=[
            pl.BlockSpec((_FBLK, _HID), lambda i: (i, 0)),
            pl.BlockSpec((_NC, _HID), lambda i: (0, 0)),
            pl.BlockSpec((_NC, 128), lambda i: (0, 0)),
        ],
        out_shape=[
            jax.ShapeDtypeStruct((_NF, _HID), jnp.float32),
            jax.ShapeDtypeStruct((_NC, _HID), jnp.float32),
            jax.ShapeDtypeStruct((_NC, 128), jnp.float32),
        ],
    )(hf0, nei, wf, par3, ones,
      p['gf_ws'], p['gf_wn'], p['gf_g'].reshape(1, -1), p['gf_b'].reshape(1, -1),
      p['f2c_W'], p['f2c_b'].reshape(1, -1))


def _parent_body(ff_ref, cfm_ref, fc_ref, cct_ref, par_ref):
    ff = ff_ref[...]
    inter = jax.lax.dot_general(
        ff, cfm_ref[...], (((1,), (1,)), ((), ())),
        preferred_element_type=jnp.float32)
    area = jnp.maximum(
        jnp.sum(ff.astype(jnp.float32), axis=1, keepdims=True), 1.0)
    ratio = inter / area
    mx = jnp.max(ratio, axis=1, keepdims=True)
    col = jax.lax.broadcasted_iota(jnp.int32, ratio.shape, 1)
    best = jnp.min(jnp.where(ratio == mx, col, 2 * _NC), axis=1, keepdims=True)
    xi = fc_ref[:, 0:1]
    yi = fc_ref[:, 1:2]
    xj = cct_ref[0:1, :]
    yj = cct_ref[1:2, :]
    d2 = (xi * xi + yi * yi) + (xj * xj + yj * yj) \
        - 2.0 * (_b16(xi) * _b16(xj) + _b16(yi) * _b16(yj))
    d2 = jnp.maximum(d2, 0.0)
    mn = jnp.min(d2, axis=1, keepdims=True)
    near = jnp.min(jnp.where(d2 == mn, col, 2 * _NC), axis=1, keepdims=True)
    par_ref[0] = jnp.where(mx > 0.5, best, near)


def _parent(ff, cfm, f_cents, c_cents_t):
    return pl.pallas_call(
        _parent_body,
        grid=(_NF // _FBLK,),
        in_specs=[
            pl.BlockSpec((_FBLK, _NF), lambda i: (i, 0)),
            pl.BlockSpec((_NC, _NF), lambda i: (0, 0)),
            pl.BlockSpec((_FBLK, 2), lambda i: (i, 0)),
            pl.BlockSpec((2, _NC), lambda i: (0, 0)),
        ],
        out_specs=pl.BlockSpec((1, _FBLK, 1), lambda i: (i, 0, 0)),
        out_shape=jax.ShapeDtypeStruct((_NF // _FBLK, _FBLK, 1), jnp.int32),
    )(ff, cfm, f_cents, c_cents_t)


def _heads_body(hf_ref, par_ref, hc1_ref, agg_ref, cnt_ref,
                ccW1_ref, ccb1_ref, ccW2_ref, ccb2_ref,
                c2fW_ref, c2fb_ref, cfW1_ref, cfb1_ref, cfW2_ref, cfb2_ref,
                pc_ref, pf_ref, hc2_ref):
    i = pl.program_id(0)

    @pl.when(i == 0)
    def _():
        hc2 = hc1_ref[...] + agg_ref[...] / jnp.maximum(cnt_ref[:, 0:1], 1.0)
        hc2_ref[...] = hc2
        t = _gelu(_mm(hc2, ccW1_ref[...]) + ccb1_ref[...])
        pc_ref[...] = jax.nn.sigmoid(_mm(t, ccW2_ref[...]) + ccb2_ref[...])

    @pl.when(i > 0)
    def _():
        p_col = par_ref[0]
        cols = jax.lax.broadcasted_iota(jnp.int32, (1, _NC), 1)
        P = jnp.where(p_col == cols, 1.0, 0.0).astype(jnp.float32)
        g = _mm(P, hc2_ref[...])
        hf2 = hf_ref[...] + _mm(g, c2fW_ref[...]) + c2fb_ref[...]
        t = _gelu(_mm(hf2, cfW1_ref[...]) + cfb1_ref[...])
        pf_ref[...] = jax.nn.sigmoid(_mm(t, cfW2_ref[...]) + cfb2_ref[...])


def _heads(hf1, par3, hc1, agg, cnt, p):
    full0 = lambda shape: pl.BlockSpec(shape, lambda i: (0,) * len(shape))
    prev = lambda i: jnp.maximum(i - 1, 0)
    pc, pf = pl.pallas_call(
        _heads_body,
        grid=(_NF // _FBLK + 1,),
        in_specs=[
            pl.BlockSpec((_FBLK, _HID), lambda i: (prev(i), 0)),
            pl.BlockSpec((1, _FBLK, 1), lambda i: (prev(i), 0, 0)),
            full0((_NC, _HID)), full0((_NC, _HID)), full0((_NC, 128)),
            full0((_HID, _HID // 2)), full0((1, _HID // 2)),
            full0((_HID // 2, 1)), full0((1, 1)),
            full0((_HID, _HID)), full0((1, _HID)),
            full0((_HID, _HID // 2)), full0((1, _HID // 2)),
            full0((_HID // 2, 1)), full0((1, 1)),
        ],
        out_specs=[
            full0((_NC, 1)),
            pl.BlockSpec((_FBLK, 1), lambda i: (prev(i), 0)),
        ],
        out_shape=[
            jax.ShapeDtypeStruct((_NC, 1), jnp.float32),
            jax.ShapeDtypeStruct((_NF, 1), jnp.float32),
        ],
        scratch_shapes=[pltpu.VMEM((_NC, _HID), jnp.float32)],
    )(hf1, par3, hc1, agg, cnt,
      p['cc_W1'], p['cc_b1'].reshape(1, -1), p['cc_W2'], p['cc_b2'].reshape(1, -1),
      p['c2f_W'], p['c2f_b'].reshape(1, -1),
      p['cf_W1'], p['cf_b1'].reshape(1, -1), p['cf_W2'], p['cf_b2'].reshape(1, -1))
    return pc, pf


def kernel(c_feats, c_cents, c_masks, f_feats, f_cents, f_masks, params):
    p = params
    f_cents_t = f_cents.T
    c_cents_t = c_cents.T
    ff = f_masks.reshape(_NF, -1)
    cfm = c_masks.reshape(_NC, -1)

    hf0, idxf, wf = _enc_knn_fine(
        f_feats, p['enc_f_W'], p['enc_f_b'].reshape(1, -1), f_cents, f_cents_t)
    nei_flat = _sc_gather(hf0, idxf[:, :_K].reshape(1, _NF * _K))
    par3 = _parent(ff, cfm, f_cents, c_cents_t)
    hc1 = _coarse_a(c_feats, c_cents, c_cents_t, p)
    nei = nei_flat.reshape(_NF, _K, _HID)
    hf1, agg, cnt = _sage_fine(hf0, nei, wf, par3, p)
    pc, pf = _heads(hf1, par3, hc1, agg, cnt, p)
    return pc.reshape(_NC), pf.reshape(_NF)


# speedup vs baseline: 3.4368x; 1.0547x over previous
import functools

import jax
import jax.numpy as jnp
from jax.experimental import pallas as pl
from jax.experimental.pallas import tpu as pltpu
from jax.experimental.pallas import tpu_sc as plsc

_HID = 256
_K = 6
_NC = 512
_NF = 4096
_FBLK = 256
_SBLK = 512

_HI = jax.lax.Precision.HIGHEST


def _gelu(x):
    return x * 0.5 * (1.0 + jax.lax.erf(x * 0.7071067811865476))


def _mm(a, b, precision=None):
    return jax.lax.dot_general(
        a, b, (((1,), (0,)), ((), ())),
        precision=precision, preferred_element_type=jnp.float32)


def _b16(x):
    return x.astype(jnp.bfloat16).astype(jnp.float32)


def _d2_block(ci_ref, ct_ref, row0):
    xi = ci_ref[:, 0:1]
    yi = ci_ref[:, 1:2]
    xj = ct_ref[0:1, :]
    yj = ct_ref[1:2, :]
    si = xi * xi + yi * yi
    sj = xj * xj + yj * yj
    d2 = si + sj - 2.0 * (_b16(xi) * _b16(xj) + _b16(yi) * _b16(yj))
    d2 = jnp.maximum(d2, 0.0)
    col = jax.lax.broadcasted_iota(jnp.int32, d2.shape, 1)
    row = jax.lax.broadcasted_iota(jnp.int32, d2.shape, 0) + row0
    return jnp.where(col == row, 1e9, d2)


def _topk_rows(d2, n_cols, k):
    col = jax.lax.broadcasted_iota(jnp.int32, d2.shape, 1)
    idxs, vals = [], []
    for _ in range(k):
        m = jnp.min(d2, axis=1, keepdims=True)
        cand = jnp.where(d2 == m, col, 2 * n_cols)
        a = jnp.min(cand, axis=1, keepdims=True)
        idxs.append(a)
        vals.append(m)
        d2 = jnp.where(col == a, 1e9, d2)
    return idxs, vals


def _topk6_fold(d2, n_cols):
    grp = n_cols // 8
    big = jnp.float32(3e9)
    sub = d2.shape[0]
    col = jax.lax.broadcasted_iota(jnp.int32, (sub, grp), 1)
    v1 = v2 = v3 = jnp.full((sub, grp), big, jnp.float32)
    c1 = c2 = c3 = jnp.zeros((sub, grp), jnp.int32)
    for g in range(8):
        v = d2[:, g * grp:(g + 1) * grp]
        c = col + g * grp
        lt1 = v < v1
        lt2 = v < v2
        lt3 = v < v3
        v3 = jnp.where(lt2, v2, jnp.where(lt3, v, v3))
        c3 = jnp.where(lt2, c2, jnp.where(lt3, c, c3))
        v2 = jnp.where(lt1, v1, jnp.where(lt2, v, v2))
        c2 = jnp.where(lt1, c1, jnp.where(lt2, c, c2))
        v1 = jnp.where(lt1, v, v1)
        c1 = jnp.where(lt1, c, c1)
    vcat = jnp.concatenate([v1, v2, v3], axis=1)
    ccat = jnp.concatenate([c1, c2, c3], axis=1)
    idxs, vals = [], []
    for _ in range(_K):
        m = jnp.min(vcat, axis=1, keepdims=True)
        cand = jnp.where(vcat == m, ccat, 2 * n_cols)
        a = jnp.min(cand, axis=1, keepdims=True)
        idxs.append(a)
        vals.append(m)
        vcat = jnp.where(ccat == a, big, vcat)
    return idxs, vals


def _knn_weights(vals):
    inv = [1.0 / jnp.maximum(jnp.sqrt(v), 1e-4) for v in vals]
    tot = functools.reduce(jnp.add, inv)
    tot = jnp.maximum(tot, 1e-8)
    return [x / tot for x in inv]


def _enc_knn_f_body(x_ref, w_ref, b_ref, ci_ref, ct_ref, ff_ref, cfm_ref,
                    cct_ref, h_ref, idx_ref, w_out_ref, par_ref):
    h_ref[...] = _gelu(_mm(x_ref[...], w_ref[...]) + b_ref[...])
    d2 = _d2_block(ci_ref, ct_ref, pl.program_id(0) * _FBLK)
    idxs, vals = _topk6_fold(d2, _NF)
    ws = _knn_weights(vals)
    for k in range(_K):
        idx_ref[:, k:k + 1] = idxs[k]
        w_out_ref[:, k:k + 1] = ws[k]
    idx_ref[:, _K:_K + 2] = jnp.zeros((_FBLK, 2), jnp.int32)
    w_out_ref[:, _K:_K + 2] = jnp.zeros((_FBLK, 2), jnp.float32)
    ff = ff_ref[...]
    inter = jax.lax.dot_general(
        ff, cfm_ref[...], (((1,), (1,)), ((), ())),
        preferred_element_type=jnp.float32)
    area = jnp.maximum(jnp.sum(ff, axis=1, keepdims=True), 1.0)
    ratio = inter / area
    mx = jnp.max(ratio, axis=1, keepdims=True)
    col = jax.lax.broadcasted_iota(jnp.int32, ratio.shape, 1)
    best = jnp.min(jnp.where(ratio == mx, col, 2 * _NC), axis=1, keepdims=True)
    xi = ci_ref[:, 0:1]
    yi = ci_ref[:, 1:2]
    xj = cct_ref[0:1, :]
    yj = cct_ref[1:2, :]
    d2c = (xi * xi + yi * yi) + (xj * xj + yj * yj) \
        - 2.0 * (_b16(xi) * _b16(xj) + _b16(yi) * _b16(yj))
    d2c = jnp.maximum(d2c, 0.0)
    mn = jnp.min(d2c, axis=1, keepdims=True)
    near = jnp.min(jnp.where(d2c == mn, col, 2 * _NC), axis=1, keepdims=True)
    par_ref[0] = jnp.where(mx > 0.5, best, near)


def _enc_knn_fine(f_feats, W, b, f_cents, f_cents_t, ff, cfm, c_cents_t):
    return pl.pallas_call(
        _enc_knn_f_body,
        grid=(_NF // _FBLK,),
        in_specs=[
            pl.BlockSpec((_FBLK, _HID), lambda i: (i, 0)),
            pl.BlockSpec((_HID, _HID), lambda i: (0, 0)),
            pl.BlockSpec((1, _HID), lambda i: (0, 0)),
            pl.BlockSpec((_FBLK, 2), lambda i: (i, 0)),
            pl.BlockSpec((2, _NF), lambda i: (0, 0)),
            pl.BlockSpec((_FBLK, _NF), lambda i: (i, 0)),
            pl.BlockSpec((_NC, _NF), lambda i: (0, 0)),
            pl.BlockSpec((2, _NC), lambda i: (0, 0)),
        ],
        out_specs=[
            pl.BlockSpec((_FBLK, _HID), lambda i: (i, 0)),
            pl.BlockSpec((_FBLK, 8), lambda i: (i, 0)),
            pl.BlockSpec((_FBLK, 8), lambda i: (i, 0)),
            pl.BlockSpec((1, _FBLK, 1), lambda i: (i, 0, 0)),
        ],
        out_shape=[
            jax.ShapeDtypeStruct((_NF, _HID), jnp.float32),
            jax.ShapeDtypeStruct((_NF, 8), jnp.int32),
            jax.ShapeDtypeStruct((_NF, 8), jnp.float32),
            jax.ShapeDtypeStruct((_NF // _FBLK, _FBLK, 1), jnp.int32),
        ],
    )(f_feats, W, b, f_cents, f_cents_t, ff, cfm, c_cents_t)


def _ln(x, g, b):
    m = x.mean(-1, keepdims=True)
    v = ((x - m) ** 2).mean(-1, keepdims=True)
    return (x - m) / jnp.sqrt(v + 1e-5) * g + b


def _coarse_a_body(cf_ref, cc_ref, cct_ref, encW_ref, encb_ref,
                   ws_ref, wn_ref, g_ref, b_ref, hc_ref):
    hc0 = _gelu(_mm(cf_ref[...], encW_ref[...]) + encb_ref[...])
    d2 = _d2_block(cc_ref, cct_ref, 0)
    idxs, vals = _topk_rows(d2, _NC, _K)
    ws = _knn_weights(vals)
    col = jax.lax.broadcasted_iota(jnp.int32, (_NC, _NC), 1)
    A = jnp.zeros((_NC, _NC), jnp.float32)
    for k in range(_K):
        A = A + jnp.where(col == idxs[k], ws[k], 0.0)
    agg = _mm(A, hc0, precision=_HI)
    out = _gelu(_mm(hc0, ws_ref[...]) + _mm(agg, wn_ref[...]))
    hc_ref[...] = hc0 + _ln(out, g_ref[...], b_ref[...])


def _coarse_a(c_feats, c_cents, c_cents_t, p):
    full = lambda shape: pl.BlockSpec(shape, lambda: (0,) * len(shape))
    return pl.pallas_call(
        _coarse_a_body,
        in_specs=[
            full((_NC, _HID)), full((_NC, 2)), full((2, _NC)),
            full((_HID, _HID)), full((1, _HID)),
            full((_HID, _HID)), full((_HID, _HID)),
            full((1, _HID)), full((1, _HID)),
        ],
        out_specs=full((_NC, _HID)),
        out_shape=jax.ShapeDtypeStruct((_NC, _HID), jnp.float32),
    )(c_feats, c_cents, c_cents_t,
      p['enc_c_W'], p['enc_c_b'].reshape(1, -1),
      p['gc_ws'], p['gc_wn'], p['gc_g'].reshape(1, -1), p['gc_b'].reshape(1, -1))


def _sc_gather(values, indices):
    M = indices.shape[1]
    D = values.shape[1]
    window = 128
    mesh = plsc.VectorSubcoreMesh(core_axis_name="core",
                                  subcore_axis_name="subcore")

    @functools.partial(
        pl.kernel,
        out_type=jax.ShapeDtypeStruct((M, D), values.dtype),
        mesh=mesh)
    def gather_kernel(x_hbm, i_hbm, o_hbm):
        def body(i_vmem, o_vmem):
            pltpu.sync_copy(x_hbm.at[i_vmem.at[0]], o_vmem)

        pltpu.emit_pipeline(
            body,
            grid=(M // window,),
            in_specs=[pl.BlockSpec((1, window), lambda i: (0, i))],
            out_specs=[pl.BlockSpec((window, D), lambda i: (i, 0))],
            core_axis_name=("core", "subcore"),
            dimension_semantics=(pltpu.PARALLEL,),
        )(i_hbm, o_hbm)

    return gather_kernel(values, indices)


def _sage_f_body(h_ref, nei_ref, w_ref, par_ref, ones_ref, ws_ref, wn_ref,
                 g_ref, b_ref, f2cW_ref, f2cb_ref, hf_ref, agg_ref, cnt_ref):
    @pl.when(pl.program_id(0) == 0)
    def _():
        agg_ref[...] = jnp.zeros_like(agg_ref)
        cnt_ref[...] = jnp.zeros_like(cnt_ref)

    agg = w_ref[:, 0:1] * nei_ref[:, 0, :]
    for k in range(1, _K):
        agg = agg + w_ref[:, k:k + 1] * nei_ref[:, k, :]
    h = h_ref[...]
    out = _gelu(_mm(h, ws_ref[...]) + _mm(agg, wn_ref[...]))
    hf1 = h + _ln(out, g_ref[...], b_ref[...])
    hf_ref[...] = hf1
    msg = _mm(hf1, f2cW_ref[...]) + f2cb_ref[...]
    p_col = par_ref[0]
    cols = jax.lax.broadcasted_iota(jnp.int32, (1, _NC), 1)
    P = jnp.where(p_col == cols, 1.0, 0.0).astype(jnp.float32)
    agg_ref[...] += jax.lax.dot_general(
        P, msg, (((0,), (0,)), ((), ())),
        preferred_element_type=jnp.float32)
    cnt_ref[...] += jax.lax.dot_general(
        P, ones_ref[...], (((0,), (0,)), ((), ())),
        preferred_element_type=jnp.float32)


def _sage_fine(hf0, nei, wf, par3, p):
    full0 = lambda shape: pl.BlockSpec(shape, lambda i: (0,) * len(shape))
    ones = jnp.ones((_FBLK, 128), jnp.float32)
    return pl.pallas_call(
        _sage_f_body,
        grid=(_NF // _FBLK,),
        in_specs=[
            pl.BlockSpec((_FBLK, _HID), lambda i: (i, 0)),
            pl.BlockSpec((_FBLK, _K, _HID), lambda i: (i, 0, 0)),
            pl.BlockSpec((_FBLK, 8), lambda i: (i, 0)),
            pl.BlockSpec((1, _FBLK, 1), lambda i: (i, 0, 0)),
            full0((_FBLK, 128)),
            full0((_HID, _HID)), full0((_HID, _HID)),
            full0((1, _HID)), full0((1, _HID)),
            full0((_HID, _HID)), full0((1, _HID)),
        ],
        out_specs=[
            pl.BlockSpec((_FBLK, _HID), lambda i: (i, 0)),
            pl.BlockSpec((_NC, _HID), lambda i: (0, 0)),
            pl.BlockSpec((_NC, 128), lambda i: (0, 0)),
        ],
        out_shape=[
            jax.ShapeDtypeStruct((_NF, _HID), jnp.float32),
            jax.ShapeDtypeStruct((_NC, _HID), jnp.float32),
            jax.ShapeDtypeStruct((_NC, 128), jnp.float32),
        ],
    )(hf0, nei, wf, par3, ones,
      p['gf_ws'], p['gf_wn'], p['gf_g'].reshape(1, -1), p['gf_b'].reshape(1, -1),
      p['f2c_W'], p['f2c_b'].reshape(1, -1))


def _parent_body(ff_ref, cfm_ref, fc_ref, cct_ref, par_ref):
    ff = ff_ref[...]
    inter = jax.lax.dot_general(
        ff, cfm_ref[...], (((1,), (1,)), ((), ())),
        preferred_element_type=jnp.float32)
    area = jnp.maximum(
        jnp.sum(ff.astype(jnp.float32), axis=1, keepdims=True), 1.0)
    ratio = inter / area
    mx = jnp.max(ratio, axis=1, keepdims=True)
    col = jax.lax.broadcasted_iota(jnp.int32, ratio.shape, 1)
    best = jnp.min(jnp.where(ratio == mx, col, 2 * _NC), axis=1, keepdims=True)
    xi = fc_ref[:, 0:1]
    yi = fc_ref[:, 1:2]
    xj = cct_ref[0:1, :]
    yj = cct_ref[1:2, :]
    d2 = (xi * xi + yi * yi) + (xj * xj + yj * yj) \
        - 2.0 * (_b16(xi) * _b16(xj) + _b16(yi) * _b16(yj))
    d2 = jnp.maximum(d2, 0.0)
    mn = jnp.min(d2, axis=1, keepdims=True)
    near = jnp.min(jnp.where(d2 == mn, col, 2 * _NC), axis=1, keepdims=True)
    par_ref[0] = jnp.where(mx > 0.5, best, near)


def _parent(ff, cfm, f_cents, c_cents_t):
    return pl.pallas_call(
        _parent_body,
        grid=(_NF // _FBLK,),
        in_specs=[
            pl.BlockSpec((_FBLK, _NF), lambda i: (i, 0)),
            pl.BlockSpec((_NC, _NF), lambda i: (0, 0)),
            pl.BlockSpec((_FBLK, 2), lambda i: (i, 0)),
            pl.BlockSpec((2, _NC), lambda i: (0, 0)),
        ],
        out_specs=pl.BlockSpec((1, _FBLK, 1), lambda i: (i, 0, 0)),
        out_shape=jax.ShapeDtypeStruct((_NF // _FBLK, _FBLK, 1), jnp.int32),
    )(ff, cfm, f_cents, c_cents_t)


def _heads_body(hf_ref, par_ref, hc1_ref, agg_ref, cnt_ref,
                ccW1_ref, ccb1_ref, ccW2_ref, ccb2_ref,
                c2fW_ref, c2fb_ref, cfW1_ref, cfb1_ref, cfW2_ref, cfb2_ref,
                pc_ref, pf_ref, hc2_ref):
    i = pl.program_id(0)

    @pl.when(i == 0)
    def _():
        hc2 = hc1_ref[...] + agg_ref[...] / jnp.maximum(cnt_ref[:, 0:1], 1.0)
        hc2_ref[...] = hc2
        t = _gelu(_mm(hc2, ccW1_ref[...]) + ccb1_ref[...])
        pc_ref[...] = jax.nn.sigmoid(_mm(t, ccW2_ref[...]) + ccb2_ref[...])

    @pl.when(i > 0)
    def _():
        p_col = par_ref[0]
        cols = jax.lax.broadcasted_iota(jnp.int32, (1, _NC), 1)
        P = jnp.where(p_col == cols, 1.0, 0.0).astype(jnp.float32)
        g = _mm(P, hc2_ref[...])
        hf2 = hf_ref[...] + _mm(g, c2fW_ref[...]) + c2fb_ref[...]
        t = _gelu(_mm(hf2, cfW1_ref[...]) + cfb1_ref[...])
        pf_ref[...] = jax.nn.sigmoid(_mm(t, cfW2_ref[...]) + cfb2_ref[...])


def _heads(hf1, par3, hc1, agg, cnt, p):
    full0 = lambda shape: pl.BlockSpec(shape, lambda i: (0,) * len(shape))
    prev = lambda i: jnp.maximum(i - 1, 0)
    pc, pf = pl.pallas_call(
        _heads_body,
        grid=(_NF // _FBLK + 1,),
        in_specs=[
            pl.BlockSpec((_FBLK, _HID), lambda i: (prev(i), 0)),
            pl.BlockSpec((1, _FBLK, 1), lambda i: (prev(i), 0, 0)),
            full0((_NC, _HID)), full0((_NC, _HID)), full0((_NC, 128)),
            full0((_HID, _HID // 2)), full0((1, _HID // 2)),
            full0((_HID // 2, 1)), full0((1, 1)),
            full0((_HID, _HID)), full0((1, _HID)),
            full0((_HID, _HID // 2)), full0((1, _HID // 2)),
            full0((_HID // 2, 1)), full0((1, 1)),
        ],
        out_specs=[
            full0((_NC, 1)),
            pl.BlockSpec((_FBLK, 1), lambda i: (prev(i), 0)),
        ],
        out_shape=[
            jax.ShapeDtypeStruct((_NC, 1), jnp.float32),
            jax.ShapeDtypeStruct((_NF, 1), jnp.float32),
        ],
        scratch_shapes=[pltpu.VMEM((_NC, _HID), jnp.float32)],
    )(hf1, par3, hc1, agg, cnt,
      p['cc_W1'], p['cc_b1'].reshape(1, -1), p['cc_W2'], p['cc_b2'].reshape(1, -1),
      p['c2f_W'], p['c2f_b'].reshape(1, -1),
      p['cf_W1'], p['cf_b1'].reshape(1, -1), p['cf_W2'], p['cf_b2'].reshape(1, -1))
    return pc, pf


def kernel(c_feats, c_cents, c_masks, f_feats, f_cents, f_masks, params):
    p = params
    f_cents_t = f_cents.T
    c_cents_t = c_cents.T
    ff = f_masks.reshape(_NF, -1)
    cfm = c_masks.reshape(_NC, -1)

    hf0, idxf, wf, par3 = _enc_knn_fine(
        f_feats, p['enc_f_W'], p['enc_f_b'].reshape(1, -1), f_cents, f_cents_t,
        ff, cfm, c_cents_t)
    nei_flat = _sc_gather(hf0, idxf[:, :_K].reshape(1, _NF * _K))
    hc1 = _coarse_a(c_feats, c_cents, c_cents_t, p)
    nei = nei_flat.reshape(_NF, _K, _HID)
    hf1, agg, cnt = _sage_fine(hf0, nei, wf, par3, p)
    pc, pf = _heads(hf1, par3, hc1, agg, cnt, p)
    return pc.reshape(_NC), pf.reshape(_NF)
